# baseline (speedup 1.0000x reference)
import functools
import jax
import jax.numpy as jnp
from jax import lax
from jax.experimental import pallas as pl
from jax.experimental.pallas import tpu as pltpu
from jax.experimental.pallas import tpu_sc as plsc

D = 768
E = 16
H = 1152
T = 4096
BT = 256
TP = T + 128

_INFO = plsc.get_sparse_core_info()
NC = _INFO.num_cores
NS = _INFO.num_subcores
NW = NC * NS
CHUNK = T // NW
NV = CHUNK // 16

_MESH = plsc.VectorSubcoreMesh(core_axis_name="c", subcore_axis_name="s")


RB = CHUNK


def _router_body(x_ref, wr_ref, idx_ref, hist_ref):
    lg = jax.lax.dot_general(
        wr_ref[...], x_ref[...], (((1,), (1,)), ((), ())),
        preferred_element_type=jnp.float32)
    mx = jnp.max(lg, axis=0, keepdims=True)
    ie = jax.lax.broadcasted_iota(jnp.int32, (E, RB), 0)
    idx = jnp.min(jnp.where(lg >= mx, ie, E), axis=0).astype(jnp.int32)
    idx_ref[0, 0, :] = idx
    onehot = (ie == idx[None, :]).astype(jnp.int32)
    hist_ref[0, 0, :] = jnp.sum(onehot, axis=1)


def _hist_body(idx_hbm, hist_hbm, idx_v, stage_v):
    wid = lax.axis_index("s") * NC + lax.axis_index("c")
    base = wid * CHUNK
    pltpu.sync_copy(idx_hbm.at[pl.ds(base, CHUNK)], idx_v)
    iota = lax.iota(jnp.int32, 16)
    hist = jnp.zeros((16,), jnp.int32)
    for j in range(NV):
        v = idx_v[pl.ds(j * 16, 16)]
        for e in range(E):
            msk = v == e
            pc = jnp.sum(msk.astype(jnp.int32))
            hist = jnp.where(iota == e, hist + pc, hist)
    stage_v[...] = hist
    pltpu.sync_copy(stage_v, hist_hbm.at[pl.ds(wid * E, E)])


def _dispatch_body(idx_hbm, x_hbm, hist_hbm,
                   xs_hbm, slot_hbm, off_hbm, cnt_hbm,
                   idx_v, histall_v, base_v, run_v, slot_v, xrows_v,
                   stage_a, stage_b, sem, sem2):
    wid = lax.axis_index("s") * NC + lax.axis_index("c")
    base = wid * CHUNK
    xcopy = pltpu.async_copy(x_hbm.at[pl.ds(base, CHUNK)], xrows_v, sem2)
    pltpu.sync_copy(idx_hbm.at[pl.ds(base, CHUNK)], idx_v)
    pltpu.sync_copy(hist_hbm, histall_v)
    iota = lax.iota(jnp.int32, 16)
    tot = jnp.zeros((16,), jnp.int32)
    pre = jnp.zeros((16,), jnp.int32)
    for w in range(NW):
        row = histall_v[pl.ds(w * E, E)]
        tot = tot + row
        before = jnp.full((16,), w, jnp.int32) < wid
        pre = jnp.where(before, pre + row, pre)
    totp = (tot + 7) & (-8)
    excl = plsc.cumsum(totp) - totp
    base_v[...] = excl + pre
    run_v[...] = jnp.zeros((16,), jnp.int32)

    @pl.when(wid == 0)
    def _():
        stage_a[...] = excl
        pltpu.sync_copy(stage_a, off_hbm)
        stage_b[...] = tot
        pltpu.sync_copy(stage_b, cnt_hbm)

    for j in range(NV):
        v = idx_v[pl.ds(j * 16, 16)]
        bl = plsc.load_gather(base_v, [v])
        rl = plsc.load_gather(run_v, [v])
        r = jnp.zeros((16,), jnp.int32)
        newrun = run_v[...]
        for e in range(E):
            msk = v == e
            c = plsc.cumsum(msk.astype(jnp.int32))
            r = jnp.where(msk, c - 1, r)
            pc = jnp.sum(msk.astype(jnp.int32))
            newrun = jnp.where(iota == e, newrun + pc, newrun)
        run_v[...] = newrun
        slot_v[pl.ds(j * 16, 16)] = bl + rl + r

    pltpu.sync_copy(slot_v, slot_hbm.at[pl.ds(base, CHUNK)])
    xcopy.wait()
    pltpu.async_copy(xrows_v, xs_hbm.at[slot_v], sem).wait()


def _unperm_body(ys_hbm, slot_hbm, out_hbm, slot_v, rows_v, sem):
    wid = lax.axis_index("s") * NC + lax.axis_index("c")
    base = wid * CHUNK
    pltpu.sync_copy(slot_hbm.at[pl.ds(base, CHUNK)], slot_v)
    pltpu.async_copy(ys_hbm.at[slot_v], rows_v, sem).wait()
    pltpu.sync_copy(rows_v, out_hbm.at[0, pl.ds(base, CHUNK)])


def _ffn_body(off_ref, cnt_ref, x_ref, wg_ref, wu_ref, wd_ref, out_ref):
    e = pl.program_id(0)
    off = off_ref[e]
    cnt = cnt_ref[e]
    end = off + cnt
    nb = (cnt + BT - 1) // BT
    wg = wg_ref[0].astype(jnp.bfloat16)
    wu = wu_ref[0].astype(jnp.bfloat16)
    wd = wd_ref[0].astype(jnp.bfloat16)

    def body(b, carry):
        row0 = off + b * BT
        row0t = pl.multiple_of(jnp.minimum(row0, TP - BT), 8)
        xb = x_ref[pl.ds(row0t, BT), :].astype(jnp.bfloat16)
        g = jax.lax.dot_general(xb, wg, (((1,), (1,)), ((), ())),
                                preferred_element_type=jnp.float32)
        u = jax.lax.dot_general(xb, wu, (((1,), (1,)), ((), ())),
                                preferred_element_type=jnp.float32)
        a = (g * jax.nn.sigmoid(g) * u).astype(jnp.bfloat16)
        y = jax.lax.dot_general(a, wd, (((1,), (1,)), ((), ())),
                                preferred_element_type=jnp.float32)
        interior = row0 + BT <= end

        @pl.when(interior)
        def _():
            out_ref[pl.ds(row0t, BT), :] = y

        @pl.when(jnp.logical_not(interior))
        def _():
            rid = row0t + jax.lax.broadcasted_iota(jnp.int32, (BT, 1), 0)
            m = (rid >= row0) & (rid < end)
            prev = out_ref[pl.ds(row0t, BT), :]
            out_ref[pl.ds(row0t, BT), :] = jnp.where(m, y, prev)

        return carry

    jax.lax.fori_loop(0, nb, body, 0)


def _grouped_ffn(xs, off, cnt, Wg, Wu, Wd):
    grid_spec = pltpu.PrefetchScalarGridSpec(
        num_scalar_prefetch=2,
        grid=(E,),
        in_specs=[
            pl.BlockSpec((TP, D), lambda e, *_: (0, 0)),
            pl.BlockSpec((1, H, D), lambda e, *_: (e, 0, 0)),
            pl.BlockSpec((1, H, D), lambda e, *_: (e, 0, 0)),
            pl.BlockSpec((1, D, H), lambda e, *_: (e, 0, 0)),
        ],
        out_specs=pl.BlockSpec((TP, D), lambda e, *_: (0, 0)),
    )
    return pl.pallas_call(
        _ffn_body,
        grid_spec=grid_spec,
        out_shape=jax.ShapeDtypeStruct((TP, D), jnp.float32),
        compiler_params=pltpu.CompilerParams(
            dimension_semantics=("arbitrary",)),
    )(off, cnt, xs, Wg, Wu, Wd)


_hist_kernel = pl.kernel(
    _hist_body,
    out_type=jax.ShapeDtypeStruct((NW * E,), jnp.int32),
    mesh=_MESH,
    compiler_params=pltpu.CompilerParams(needs_layout_passes=False),
    scratch_types=[
        pltpu.VMEM((CHUNK,), jnp.int32),
        pltpu.VMEM((E,), jnp.int32),
    ],
)

_dispatch_kernel = pl.kernel(
    _dispatch_body,
    out_type=(
        jax.ShapeDtypeStruct((TP, D), jnp.float32),
        jax.ShapeDtypeStruct((T,), jnp.int32),
        jax.ShapeDtypeStruct((E,), jnp.int32),
        jax.ShapeDtypeStruct((E,), jnp.int32),
    ),
    mesh=_MESH,
    compiler_params=pltpu.CompilerParams(needs_layout_passes=False),
    scratch_types=[
        pltpu.VMEM((CHUNK,), jnp.int32),
        pltpu.VMEM((NW * E,), jnp.int32),
        pltpu.VMEM((E,), jnp.int32),
        pltpu.VMEM((E,), jnp.int32),
        pltpu.VMEM((CHUNK,), jnp.int32),
        pltpu.VMEM((CHUNK, D), jnp.float32),
        pltpu.VMEM((E,), jnp.int32),
        pltpu.VMEM((E,), jnp.int32),
        pltpu.SemaphoreType.DMA,
        pltpu.SemaphoreType.DMA,
    ],
)

_unperm_kernel = pl.kernel(
    _unperm_body,
    out_type=jax.ShapeDtypeStruct((1, T, D), jnp.float32),
    mesh=_MESH,
    compiler_params=pltpu.CompilerParams(needs_layout_passes=False),
    scratch_types=[
        pltpu.VMEM((CHUNK,), jnp.int32),
        pltpu.VMEM((CHUNK, D), jnp.float32),
        pltpu.SemaphoreType.DMA,
    ],
)


@jax.jit
def kernel(x, Wr, Wg, Wu, Wd):
    xf = x.reshape(T, D)
    idx3, hist3 = pl.pallas_call(
        _router_body,
        grid=(T // RB,),
        in_specs=[
            pl.BlockSpec((RB, D), lambda i: (i, 0)),
            pl.BlockSpec((E, D), lambda i: (0, 0)),
        ],
        out_specs=[
            pl.BlockSpec((1, 1, RB), lambda i: (i, 0, 0)),
            pl.BlockSpec((1, 1, E), lambda i: (i, 0, 0)),
        ],
        out_shape=[
            jax.ShapeDtypeStruct((NW, 1, RB), jnp.int32),
            jax.ShapeDtypeStruct((NW, 1, E), jnp.int32),
        ],
    )(xf, Wr)
    idx = idx3.reshape(T)
    hist = hist3.reshape(NW * E)

    xs, slot, off, cnt = _dispatch_kernel(idx, xf, hist)
    ys = _grouped_ffn(xs, off, cnt, Wg, Wu, Wd)
    return _unperm_kernel(ys, slot)


# speedup vs baseline: 1.0771x; 1.0771x over previous
import functools
import jax
import jax.numpy as jnp
from jax import lax
from jax.experimental import pallas as pl
from jax.experimental.pallas import tpu as pltpu
from jax.experimental.pallas import tpu_sc as plsc

D = 768
E = 16
H = 1152
T = 4096
BT = 256
TP = T + 128

_INFO = plsc.get_sparse_core_info()
NC = _INFO.num_cores
NS = _INFO.num_subcores
NW = NC * NS
CHUNK = T // NW
NV = CHUNK // 16

_MESH = plsc.VectorSubcoreMesh(core_axis_name="c", subcore_axis_name="s")


RB = 2 * CHUNK


def _router_body(x_ref, wr_ref, idx_ref, hist_ref):
    lg = jax.lax.dot_general(
        wr_ref[...], x_ref[...], (((1,), (1,)), ((), ())),
        preferred_element_type=jnp.float32)
    mx = jnp.max(lg, axis=0, keepdims=True)
    ie = jax.lax.broadcasted_iota(jnp.int32, (E, RB), 0)
    idx = jnp.min(jnp.where(lg >= mx, ie, E), axis=0).astype(jnp.int32)
    idx_ref[0, 0, :] = idx
    onehot = (ie == idx[None, :]).astype(jnp.int32)
    hist_ref[0, 0, :] = jnp.sum(onehot[:, :CHUNK], axis=1)
    hist_ref[0, 1, :] = jnp.sum(onehot[:, CHUNK:], axis=1)


def _hist_body(idx_hbm, hist_hbm, idx_v, stage_v):
    wid = lax.axis_index("s") * NC + lax.axis_index("c")
    base = wid * CHUNK
    pltpu.sync_copy(idx_hbm.at[pl.ds(base, CHUNK)], idx_v)
    iota = lax.iota(jnp.int32, 16)
    hist = jnp.zeros((16,), jnp.int32)
    for j in range(NV):
        v = idx_v[pl.ds(j * 16, 16)]
        for e in range(E):
            msk = v == e
            pc = jnp.sum(msk.astype(jnp.int32))
            hist = jnp.where(iota == e, hist + pc, hist)
    stage_v[...] = hist
    pltpu.sync_copy(stage_v, hist_hbm.at[pl.ds(wid * E, E)])


def _dispatch_body(idx_hbm, x_hbm, hist_hbm,
                   xs_hbm, slot_hbm, off_hbm, cnt_hbm,
                   idx_v, histall_v, base_v, run_v, slot_v, xrows_v,
                   stage_a, stage_b, sem, sem2):
    wid = lax.axis_index("s") * NC + lax.axis_index("c")
    base = wid * CHUNK
    xcopy = pltpu.async_copy(x_hbm.at[pl.ds(base, CHUNK)], xrows_v, sem2)
    pltpu.sync_copy(idx_hbm.at[pl.ds(base, CHUNK)], idx_v)
    pltpu.sync_copy(hist_hbm, histall_v)
    iota = lax.iota(jnp.int32, 16)
    tot = jnp.zeros((16,), jnp.int32)
    pre = jnp.zeros((16,), jnp.int32)
    for w in range(NW):
        row = histall_v[pl.ds(w * E, E)]
        tot = tot + row
        before = jnp.full((16,), w, jnp.int32) < wid
        pre = jnp.where(before, pre + row, pre)
    totp = (tot + 7) & (-8)
    excl = plsc.cumsum(totp) - totp
    base_v[...] = excl + pre
    run_v[...] = jnp.zeros((16,), jnp.int32)

    @pl.when(wid == 0)
    def _():
        stage_a[...] = excl
        pltpu.sync_copy(stage_a, off_hbm)
        stage_b[...] = tot
        pltpu.sync_copy(stage_b, cnt_hbm)

    for j in range(NV):
        v = idx_v[pl.ds(j * 16, 16)]
        bl = plsc.load_gather(base_v, [v])
        rl = plsc.load_gather(run_v, [v])
        r = jnp.zeros((16,), jnp.int32)
        newrun = run_v[...]
        for e in range(E):
            msk = v == e
            c = plsc.cumsum(msk.astype(jnp.int32))
            r = jnp.where(msk, c - 1, r)
            pc = jnp.sum(msk.astype(jnp.int32))
            newrun = jnp.where(iota == e, newrun + pc, newrun)
        run_v[...] = newrun
        slot_v[pl.ds(j * 16, 16)] = bl + rl + r

    pltpu.sync_copy(slot_v, slot_hbm.at[pl.ds(base, CHUNK)])
    xcopy.wait()
    pltpu.async_copy(xrows_v, xs_hbm.at[slot_v], sem).wait()


def _unperm_body(ys_hbm, slot_hbm, out_hbm, slot_v, rows_v, sem):
    wid = lax.axis_index("s") * NC + lax.axis_index("c")
    base = wid * CHUNK
    pltpu.sync_copy(slot_hbm.at[pl.ds(base, CHUNK)], slot_v)
    pltpu.async_copy(ys_hbm.at[slot_v], rows_v, sem).wait()
    pltpu.sync_copy(rows_v, out_hbm.at[0, pl.ds(base, CHUNK)])


def _ffn_body(off_ref, cnt_ref, x_ref, wg_ref, wu_ref, wd_ref, out_ref):
    e = pl.program_id(0)
    off = off_ref[e]
    cnt = cnt_ref[e]
    end = off + cnt
    nb = (cnt + BT - 1) // BT
    wg = wg_ref[0].astype(jnp.bfloat16)
    wu = wu_ref[0].astype(jnp.bfloat16)
    wd = wd_ref[0].astype(jnp.bfloat16)

    def body(b, carry):
        row0 = off + b * BT
        row0t = pl.multiple_of(jnp.minimum(row0, TP - BT), 8)
        xb = x_ref[pl.ds(row0t, BT), :].astype(jnp.bfloat16)
        g = jax.lax.dot_general(xb, wg, (((1,), (1,)), ((), ())),
                                preferred_element_type=jnp.float32)
        u = jax.lax.dot_general(xb, wu, (((1,), (1,)), ((), ())),
                                preferred_element_type=jnp.float32)
        a = (g * jax.nn.sigmoid(g) * u).astype(jnp.bfloat16)
        y = jax.lax.dot_general(a, wd, (((1,), (1,)), ((), ())),
                                preferred_element_type=jnp.float32)
        interior = row0 + BT <= end

        @pl.when(interior)
        def _():
            out_ref[pl.ds(row0t, BT), :] = y

        @pl.when(jnp.logical_not(interior))
        def _():
            rid = row0t + jax.lax.broadcasted_iota(jnp.int32, (BT, 1), 0)
            m = (rid >= row0) & (rid < end)
            prev = out_ref[pl.ds(row0t, BT), :]
            out_ref[pl.ds(row0t, BT), :] = jnp.where(m, y, prev)

        return carry

    jax.lax.fori_loop(0, nb, body, 0)


def _grouped_ffn(xs, off, cnt, Wg, Wu, Wd):
    grid_spec = pltpu.PrefetchScalarGridSpec(
        num_scalar_prefetch=2,
        grid=(E,),
        in_specs=[
            pl.BlockSpec((TP, D), lambda e, *_: (0, 0)),
            pl.BlockSpec((1, H, D), lambda e, *_: (e, 0, 0)),
            pl.BlockSpec((1, H, D), lambda e, *_: (e, 0, 0)),
            pl.BlockSpec((1, D, H), lambda e, *_: (e, 0, 0)),
        ],
        out_specs=pl.BlockSpec((TP, D), lambda e, *_: (0, 0)),
    )
    return pl.pallas_call(
        _ffn_body,
        grid_spec=grid_spec,
        out_shape=jax.ShapeDtypeStruct((TP, D), jnp.float32),
        compiler_params=pltpu.CompilerParams(
            dimension_semantics=("arbitrary",)),
    )(off, cnt, xs, Wg, Wu, Wd)


_hist_kernel = pl.kernel(
    _hist_body,
    out_type=jax.ShapeDtypeStruct((NW * E,), jnp.int32),
    mesh=_MESH,
    compiler_params=pltpu.CompilerParams(needs_layout_passes=False),
    scratch_types=[
        pltpu.VMEM((CHUNK,), jnp.int32),
        pltpu.VMEM((E,), jnp.int32),
    ],
)

_dispatch_kernel = pl.kernel(
    _dispatch_body,
    out_type=(
        jax.ShapeDtypeStruct((TP, D), jnp.float32),
        jax.ShapeDtypeStruct((T,), jnp.int32),
        jax.ShapeDtypeStruct((E,), jnp.int32),
        jax.ShapeDtypeStruct((E,), jnp.int32),
    ),
    mesh=_MESH,
    compiler_params=pltpu.CompilerParams(needs_layout_passes=False),
    scratch_types=[
        pltpu.VMEM((CHUNK,), jnp.int32),
        pltpu.VMEM((NW * E,), jnp.int32),
        pltpu.VMEM((E,), jnp.int32),
        pltpu.VMEM((E,), jnp.int32),
        pltpu.VMEM((CHUNK,), jnp.int32),
        pltpu.VMEM((CHUNK, D), jnp.float32),
        pltpu.VMEM((E,), jnp.int32),
        pltpu.VMEM((E,), jnp.int32),
        pltpu.SemaphoreType.DMA,
        pltpu.SemaphoreType.DMA,
    ],
)

_unperm_kernel = pl.kernel(
    _unperm_body,
    out_type=jax.ShapeDtypeStruct((1, T, D), jnp.float32),
    mesh=_MESH,
    compiler_params=pltpu.CompilerParams(needs_layout_passes=False),
    scratch_types=[
        pltpu.VMEM((CHUNK,), jnp.int32),
        pltpu.VMEM((CHUNK, D), jnp.float32),
        pltpu.SemaphoreType.DMA,
    ],
)


@jax.jit
def kernel(x, Wr, Wg, Wu, Wd):
    xf = x.reshape(T, D)
    idx3, hist3 = pl.pallas_call(
        _router_body,
        grid=(T // RB,),
        in_specs=[
            pl.BlockSpec((RB, D), lambda i: (i, 0)),
            pl.BlockSpec((E, D), lambda i: (0, 0)),
        ],
        out_specs=[
            pl.BlockSpec((1, 1, RB), lambda i: (i, 0, 0)),
            pl.BlockSpec((1, 2, E), lambda i: (i, 0, 0)),
        ],
        out_shape=[
            jax.ShapeDtypeStruct((T // RB, 1, RB), jnp.int32),
            jax.ShapeDtypeStruct((T // RB, 2, E), jnp.int32),
        ],
    )(xf, Wr)
    idx = idx3.reshape(T)
    hist = hist3.reshape(NW * E)

    xs, slot, off, cnt = _dispatch_kernel(idx, xf, hist)
    ys = _grouped_ffn(xs, off, cnt, Wg, Wu, Wd)
    return _unperm_kernel(ys, slot)


# speedup vs baseline: 1.0788x; 1.0016x over previous
import functools
import jax
import jax.numpy as jnp
from jax import lax
from jax.experimental import pallas as pl
from jax.experimental.pallas import tpu as pltpu
from jax.experimental.pallas import tpu_sc as plsc

D = 768
E = 16
H = 1152
T = 4096
BT = 256
TP = T + 128

_INFO = plsc.get_sparse_core_info()
NC = _INFO.num_cores
NS = _INFO.num_subcores
NW = NC * NS
CHUNK = T // NW
NV = CHUNK // 16

_MESH = plsc.VectorSubcoreMesh(core_axis_name="c", subcore_axis_name="s")


RB = 2 * CHUNK


def _router_body(x_ref, wr_ref, idx_ref, hist_ref):
    lg = jax.lax.dot_general(
        wr_ref[...], x_ref[...], (((1,), (1,)), ((), ())),
        preferred_element_type=jnp.float32)
    mx = jnp.max(lg, axis=0, keepdims=True)
    ie = jax.lax.broadcasted_iota(jnp.int32, (E, RB), 0)
    idx = jnp.min(jnp.where(lg >= mx, ie, E), axis=0).astype(jnp.int32)
    idx_ref[0, 0, :] = idx
    onehot = (ie == idx[None, :]).astype(jnp.int32)
    hist_ref[0, 0, :] = jnp.sum(onehot[:, :CHUNK], axis=1)
    hist_ref[0, 1, :] = jnp.sum(onehot[:, CHUNK:], axis=1)


def _hist_body(idx_hbm, hist_hbm, idx_v, stage_v):
    wid = lax.axis_index("s") * NC + lax.axis_index("c")
    base = wid * CHUNK
    pltpu.sync_copy(idx_hbm.at[pl.ds(base, CHUNK)], idx_v)
    iota = lax.iota(jnp.int32, 16)
    hist = jnp.zeros((16,), jnp.int32)
    for j in range(NV):
        v = idx_v[pl.ds(j * 16, 16)]
        for e in range(E):
            msk = v == e
            pc = jnp.sum(msk.astype(jnp.int32))
            hist = jnp.where(iota == e, hist + pc, hist)
    stage_v[...] = hist
    pltpu.sync_copy(stage_v, hist_hbm.at[pl.ds(wid * E, E)])


def _dispatch_body(idx_hbm, x_hbm, hist_hbm,
                   xs_hbm, slot_hbm, off_hbm, cnt_hbm,
                   idx_v, histall_v, base_v, run_v, slot_v, xrows_v,
                   stage_a, stage_b, sem, sem2):
    wid = lax.axis_index("s") * NC + lax.axis_index("c")
    base = wid * CHUNK
    xcopy = pltpu.async_copy(x_hbm.at[pl.ds(base, CHUNK)], xrows_v, sem2)
    pltpu.sync_copy(idx_hbm.at[pl.ds(base, CHUNK)], idx_v)
    pltpu.sync_copy(hist_hbm, histall_v)
    iota = lax.iota(jnp.int32, 16)
    tot = jnp.zeros((16,), jnp.int32)
    pre = jnp.zeros((16,), jnp.int32)
    for w in range(NW):
        row = histall_v[pl.ds(w * E, E)]
        tot = tot + row
        before = jnp.full((16,), w, jnp.int32) < wid
        pre = jnp.where(before, pre + row, pre)
    totp = (tot + 7) & (-8)
    excl = plsc.cumsum(totp) - totp
    base_v[...] = excl + pre
    run_v[...] = jnp.zeros((16,), jnp.int32)

    @pl.when(wid == 0)
    def _():
        stage_a[...] = excl
        pltpu.sync_copy(stage_a, off_hbm)
        stage_b[...] = tot
        pltpu.sync_copy(stage_b, cnt_hbm)

    for j in range(NV):
        v = idx_v[pl.ds(j * 16, 16)]
        bl = plsc.load_gather(base_v, [v])
        rl = plsc.load_gather(run_v, [v])
        r = jnp.zeros((16,), jnp.int32)
        newrun = run_v[...]
        for e in range(E):
            msk = v == e
            c = plsc.cumsum(msk.astype(jnp.int32))
            r = jnp.where(msk, c - 1, r)
            pc = jnp.sum(msk.astype(jnp.int32))
            newrun = jnp.where(iota == e, newrun + pc, newrun)
        run_v[...] = newrun
        slot_v[pl.ds(j * 16, 16)] = bl + rl + r

    pltpu.sync_copy(slot_v, slot_hbm.at[pl.ds(base, CHUNK)])
    xcopy.wait()
    pltpu.async_copy(xrows_v, xs_hbm.at[slot_v], sem).wait()


HC = CHUNK // 2


def _unperm_body(ys_hbm, slot_hbm, out_hbm, slot_v, rows_a, rows_b,
                 sem_a, sem_b, sem_w):
    wid = lax.axis_index("s") * NC + lax.axis_index("c")
    base = wid * CHUNK
    pltpu.sync_copy(slot_hbm.at[pl.ds(base, CHUNK)], slot_v)
    g0 = pltpu.async_copy(ys_hbm.at[slot_v.at[pl.ds(0, HC)]], rows_a, sem_a)
    g1 = pltpu.async_copy(ys_hbm.at[slot_v.at[pl.ds(HC, HC)]], rows_b, sem_b)
    g0.wait()
    w0 = pltpu.async_copy(rows_a, out_hbm.at[0, pl.ds(base, HC)], sem_w)
    g1.wait()
    w1 = pltpu.async_copy(rows_b, out_hbm.at[0, pl.ds(base + HC, HC)], sem_w)
    w0.wait()
    w1.wait()


def _ffn_body(off_ref, cnt_ref, x_ref, wg_ref, wu_ref, wd_ref, out_ref):
    e = pl.program_id(0)
    off = off_ref[e]
    cnt = cnt_ref[e]
    end = off + cnt
    nb = (cnt + BT - 1) // BT
    wg = wg_ref[0].astype(jnp.bfloat16)
    wu = wu_ref[0].astype(jnp.bfloat16)
    wd = wd_ref[0].astype(jnp.bfloat16)

    def body(b, carry):
        row0 = off + b * BT
        row0t = pl.multiple_of(jnp.minimum(row0, TP - BT), 8)
        xb = x_ref[pl.ds(row0t, BT), :].astype(jnp.bfloat16)
        g = jax.lax.dot_general(xb, wg, (((1,), (1,)), ((), ())),
                                preferred_element_type=jnp.float32)
        u = jax.lax.dot_general(xb, wu, (((1,), (1,)), ((), ())),
                                preferred_element_type=jnp.float32)
        a = (g * jax.nn.sigmoid(g) * u).astype(jnp.bfloat16)
        y = jax.lax.dot_general(a, wd, (((1,), (1,)), ((), ())),
                                preferred_element_type=jnp.float32)
        interior = row0 + BT <= end

        @pl.when(interior)
        def _():
            out_ref[pl.ds(row0t, BT), :] = y

        @pl.when(jnp.logical_not(interior))
        def _():
            rid = row0t + jax.lax.broadcasted_iota(jnp.int32, (BT, 1), 0)
            m = (rid >= row0) & (rid < end)
            prev = out_ref[pl.ds(row0t, BT), :]
            out_ref[pl.ds(row0t, BT), :] = jnp.where(m, y, prev)

        return carry

    jax.lax.fori_loop(0, nb, body, 0)


def _grouped_ffn(xs, off, cnt, Wg, Wu, Wd):
    grid_spec = pltpu.PrefetchScalarGridSpec(
        num_scalar_prefetch=2,
        grid=(E,),
        in_specs=[
            pl.BlockSpec((TP, D), lambda e, *_: (0, 0)),
            pl.BlockSpec((1, H, D), lambda e, *_: (e, 0, 0)),
            pl.BlockSpec((1, H, D), lambda e, *_: (e, 0, 0)),
            pl.BlockSpec((1, D, H), lambda e, *_: (e, 0, 0)),
        ],
        out_specs=pl.BlockSpec((TP, D), lambda e, *_: (0, 0)),
    )
    return pl.pallas_call(
        _ffn_body,
        grid_spec=grid_spec,
        out_shape=jax.ShapeDtypeStruct((TP, D), jnp.float32),
        compiler_params=pltpu.CompilerParams(
            dimension_semantics=("arbitrary",)),
    )(off, cnt, xs, Wg, Wu, Wd)


_hist_kernel = pl.kernel(
    _hist_body,
    out_type=jax.ShapeDtypeStruct((NW * E,), jnp.int32),
    mesh=_MESH,
    compiler_params=pltpu.CompilerParams(needs_layout_passes=False),
    scratch_types=[
        pltpu.VMEM((CHUNK,), jnp.int32),
        pltpu.VMEM((E,), jnp.int32),
    ],
)

_dispatch_kernel = pl.kernel(
    _dispatch_body,
    out_type=(
        jax.ShapeDtypeStruct((TP, D), jnp.float32),
        jax.ShapeDtypeStruct((T,), jnp.int32),
        jax.ShapeDtypeStruct((E,), jnp.int32),
        jax.ShapeDtypeStruct((E,), jnp.int32),
    ),
    mesh=_MESH,
    compiler_params=pltpu.CompilerParams(needs_layout_passes=False),
    scratch_types=[
        pltpu.VMEM((CHUNK,), jnp.int32),
        pltpu.VMEM((NW * E,), jnp.int32),
        pltpu.VMEM((E,), jnp.int32),
        pltpu.VMEM((E,), jnp.int32),
        pltpu.VMEM((CHUNK,), jnp.int32),
        pltpu.VMEM((CHUNK, D), jnp.float32),
        pltpu.VMEM((E,), jnp.int32),
        pltpu.VMEM((E,), jnp.int32),
        pltpu.SemaphoreType.DMA,
        pltpu.SemaphoreType.DMA,
    ],
)

_unperm_kernel = pl.kernel(
    _unperm_body,
    out_type=jax.ShapeDtypeStruct((1, T, D), jnp.float32),
    mesh=_MESH,
    compiler_params=pltpu.CompilerParams(needs_layout_passes=False),
    scratch_types=[
        pltpu.VMEM((CHUNK,), jnp.int32),
        pltpu.VMEM((HC, D), jnp.float32),
        pltpu.VMEM((HC, D), jnp.float32),
        pltpu.SemaphoreType.DMA,
        pltpu.SemaphoreType.DMA,
        pltpu.SemaphoreType.DMA,
    ],
)


@jax.jit
def kernel(x, Wr, Wg, Wu, Wd):
    xf = x.reshape(T, D)
    idx3, hist3 = pl.pallas_call(
        _router_body,
        grid=(T // RB,),
        in_specs=[
            pl.BlockSpec((RB, D), lambda i: (i, 0)),
            pl.BlockSpec((E, D), lambda i: (0, 0)),
        ],
        out_specs=[
            pl.BlockSpec((1, 1, RB), lambda i: (i, 0, 0)),
            pl.BlockSpec((1, 2, E), lambda i: (i, 0, 0)),
        ],
        out_shape=[
            jax.ShapeDtypeStruct((T // RB, 1, RB), jnp.int32),
            jax.ShapeDtypeStruct((T // RB, 2, E), jnp.int32),
        ],
    )(xf, Wr)
    idx = idx3.reshape(T)
    hist = hist3.reshape(NW * E)

    xs, slot, off, cnt = _dispatch_kernel(idx, xf, hist)
    ys = _grouped_ffn(xs, off, cnt, Wg, Wu, Wd)
    return _unperm_kernel(ys, slot)


# speedup vs baseline: 1.0789x; 1.0001x over previous
import jax
import jax.numpy as jnp
from jax import lax
from jax.experimental import pallas as pl
from jax.experimental.pallas import tpu as pltpu
from jax.experimental.pallas import tpu_sc as plsc

D = 768
E = 16
H = 1152
T = 4096
BT = 256
TP = T + 128

_INFO = plsc.get_sparse_core_info()
NC = _INFO.num_cores
NS = _INFO.num_subcores
NW = NC * NS
CHUNK = T // NW
NV = CHUNK // 16

_MESH = plsc.VectorSubcoreMesh(core_axis_name="c", subcore_axis_name="s")


RB = 2 * CHUNK


def _router_body(x_ref, wr_ref, idx_ref, hist_ref):
    lg = jax.lax.dot_general(
        wr_ref[...], x_ref[...], (((1,), (1,)), ((), ())),
        preferred_element_type=jnp.float32)
    mx = jnp.max(lg, axis=0, keepdims=True)
    ie = jax.lax.broadcasted_iota(jnp.int32, (E, RB), 0)
    idx = jnp.min(jnp.where(lg >= mx, ie, E), axis=0).astype(jnp.int32)
    idx_ref[0, 0, :] = idx
    onehot = (ie == idx[None, :]).astype(jnp.int32)
    hist_ref[0, 0, :] = jnp.sum(onehot[:, :CHUNK], axis=1)
    hist_ref[0, 1, :] = jnp.sum(onehot[:, CHUNK:], axis=1)


def _dispatch_body(idx_hbm, x_hbm, hist_hbm,
                   xs_hbm, slot_hbm, off_hbm, cnt_hbm,
                   idx_v, histall_v, base_v, run_v, slot_v, xrows_v,
                   stage_a, stage_b, sem, sem2):
    wid = lax.axis_index("s") * NC + lax.axis_index("c")
    base = wid * CHUNK
    xcopy = pltpu.async_copy(x_hbm.at[pl.ds(base, CHUNK)], xrows_v, sem2)
    pltpu.sync_copy(idx_hbm.at[pl.ds(base, CHUNK)], idx_v)
    pltpu.sync_copy(hist_hbm, histall_v)
    iota = lax.iota(jnp.int32, 16)
    tot = jnp.zeros((16,), jnp.int32)
    pre = jnp.zeros((16,), jnp.int32)
    for w in range(NW):
        row = histall_v[pl.ds(w * E, E)]
        tot = tot + row
        before = jnp.full((16,), w, jnp.int32) < wid
        pre = jnp.where(before, pre + row, pre)
    totp = (tot + 7) & (-8)
    excl = plsc.cumsum(totp) - totp
    base_v[...] = excl + pre
    run_v[...] = jnp.zeros((16,), jnp.int32)

    @pl.when(wid == 0)
    def _():
        stage_a[...] = excl
        pltpu.sync_copy(stage_a, off_hbm)
        stage_b[...] = tot
        pltpu.sync_copy(stage_b, cnt_hbm)

    for j in range(NV):
        v = idx_v[pl.ds(j * 16, 16)]
        bl = plsc.load_gather(base_v, [v])
        rl = plsc.load_gather(run_v, [v])
        r = jnp.zeros((16,), jnp.int32)
        newrun = run_v[...]
        for e in range(E):
            msk = v == e
            c = plsc.cumsum(msk.astype(jnp.int32))
            r = jnp.where(msk, c - 1, r)
            pc = jnp.sum(msk.astype(jnp.int32))
            newrun = jnp.where(iota == e, newrun + pc, newrun)
        run_v[...] = newrun
        slot_v[pl.ds(j * 16, 16)] = bl + rl + r

    pltpu.sync_copy(slot_v, slot_hbm.at[pl.ds(base, CHUNK)])
    xcopy.wait()
    pltpu.async_copy(xrows_v, xs_hbm.at[slot_v], sem).wait()


HC = CHUNK // 2


def _unperm_body(ys_hbm, slot_hbm, out_hbm, slot_v, rows_a, rows_b,
                 sem_a, sem_b, sem_w):
    wid = lax.axis_index("s") * NC + lax.axis_index("c")
    base = wid * CHUNK
    pltpu.sync_copy(slot_hbm.at[pl.ds(base, CHUNK)], slot_v)
    g0 = pltpu.async_copy(ys_hbm.at[slot_v.at[pl.ds(0, HC)]], rows_a, sem_a)
    g1 = pltpu.async_copy(ys_hbm.at[slot_v.at[pl.ds(HC, HC)]], rows_b, sem_b)
    g0.wait()
    w0 = pltpu.async_copy(rows_a, out_hbm.at[0, pl.ds(base, HC)], sem_w)
    g1.wait()
    w1 = pltpu.async_copy(rows_b, out_hbm.at[0, pl.ds(base + HC, HC)], sem_w)
    w0.wait()
    w1.wait()


def _ffn_body(off_ref, cnt_ref, x_ref, wg_ref, wu_ref, wd_ref, out_ref):
    e = pl.program_id(0)
    off = off_ref[e]
    cnt = cnt_ref[e]
    end = off + cnt
    nb = (cnt + BT - 1) // BT
    wg = wg_ref[0].astype(jnp.bfloat16)
    wu = wu_ref[0].astype(jnp.bfloat16)
    wd = wd_ref[0].astype(jnp.bfloat16)

    def body(b, carry):
        row0 = off + b * BT
        row0t = pl.multiple_of(jnp.minimum(row0, TP - BT), 8)
        xb = x_ref[pl.ds(row0t, BT), :].astype(jnp.bfloat16)
        g = jax.lax.dot_general(xb, wg, (((1,), (1,)), ((), ())),
                                preferred_element_type=jnp.float32)
        u = jax.lax.dot_general(xb, wu, (((1,), (1,)), ((), ())),
                                preferred_element_type=jnp.float32)
        a = (g * jax.nn.sigmoid(g) * u).astype(jnp.bfloat16)
        y = jax.lax.dot_general(a, wd, (((1,), (1,)), ((), ())),
                                preferred_element_type=jnp.float32)
        interior = row0 + BT <= end

        @pl.when(interior)
        def _():
            out_ref[pl.ds(row0t, BT), :] = y

        @pl.when(jnp.logical_not(interior))
        def _():
            rid = row0t + jax.lax.broadcasted_iota(jnp.int32, (BT, 1), 0)
            m = (rid >= row0) & (rid < end)
            prev = out_ref[pl.ds(row0t, BT), :]
            out_ref[pl.ds(row0t, BT), :] = jnp.where(m, y, prev)

        return carry

    jax.lax.fori_loop(0, nb, body, 0)


def _grouped_ffn(xs, off, cnt, Wg, Wu, Wd):
    grid_spec = pltpu.PrefetchScalarGridSpec(
        num_scalar_prefetch=2,
        grid=(E,),
        in_specs=[
            pl.BlockSpec((TP, D), lambda e, *_: (0, 0)),
            pl.BlockSpec((1, H, D), lambda e, *_: (e, 0, 0)),
            pl.BlockSpec((1, H, D), lambda e, *_: (e, 0, 0)),
            pl.BlockSpec((1, D, H), lambda e, *_: (e, 0, 0)),
        ],
        out_specs=pl.BlockSpec((TP, D), lambda e, *_: (0, 0)),
    )
    return pl.pallas_call(
        _ffn_body,
        grid_spec=grid_spec,
        out_shape=jax.ShapeDtypeStruct((TP, D), jnp.float32),
        compiler_params=pltpu.CompilerParams(
            dimension_semantics=("arbitrary",)),
    )(off, cnt, xs, Wg, Wu, Wd)


_dispatch_kernel = pl.kernel(
    _dispatch_body,
    out_type=(
        jax.ShapeDtypeStruct((TP, D), jnp.float32),
        jax.ShapeDtypeStruct((T,), jnp.int32),
        jax.ShapeDtypeStruct((E,), jnp.int32),
        jax.ShapeDtypeStruct((E,), jnp.int32),
    ),
    mesh=_MESH,
    compiler_params=pltpu.CompilerParams(needs_layout_passes=False),
    scratch_types=[
        pltpu.VMEM((CHUNK,), jnp.int32),
        pltpu.VMEM((NW * E,), jnp.int32),
        pltpu.VMEM((E,), jnp.int32),
        pltpu.VMEM((E,), jnp.int32),
        pltpu.VMEM((CHUNK,), jnp.int32),
        pltpu.VMEM((CHUNK, D), jnp.float32),
        pltpu.VMEM((E,), jnp.int32),
        pltpu.VMEM((E,), jnp.int32),
        pltpu.SemaphoreType.DMA,
        pltpu.SemaphoreType.DMA,
    ],
)

_unperm_kernel = pl.kernel(
    _unperm_body,
    out_type=jax.ShapeDtypeStruct((1, T, D), jnp.float32),
    mesh=_MESH,
    compiler_params=pltpu.CompilerParams(needs_layout_passes=False),
    scratch_types=[
        pltpu.VMEM((CHUNK,), jnp.int32),
        pltpu.VMEM((HC, D), jnp.float32),
        pltpu.VMEM((HC, D), jnp.float32),
        pltpu.SemaphoreType.DMA,
        pltpu.SemaphoreType.DMA,
        pltpu.SemaphoreType.DMA,
    ],
)


@jax.jit
def kernel(x, Wr, Wg, Wu, Wd):
    xf = x.reshape(T, D)
    idx3, hist3 = pl.pallas_call(
        _router_body,
        grid=(T // RB,),
        in_specs=[
            pl.BlockSpec((RB, D), lambda i: (i, 0)),
            pl.BlockSpec((E, D), lambda i: (0, 0)),
        ],
        out_specs=[
            pl.BlockSpec((1, 1, RB), lambda i: (i, 0, 0)),
            pl.BlockSpec((1, 2, E), lambda i: (i, 0, 0)),
        ],
        out_shape=[
            jax.ShapeDtypeStruct((T // RB, 1, RB), jnp.int32),
            jax.ShapeDtypeStruct((T // RB, 2, E), jnp.int32),
        ],
    )(xf, Wr)
    idx = idx3.reshape(T)
    hist = hist3.reshape(NW * E)

    xs, slot, off, cnt = _dispatch_kernel(idx, xf, hist)
    ys = _grouped_ffn(xs, off, cnt, Wg, Wu, Wd)
    return _unperm_kernel(ys, slot)
